# baseline (speedup 1.0000x reference)
import functools

import jax
import jax.numpy as jnp
from jax import lax
from jax.experimental import pallas as pl
from jax.experimental.pallas import tpu as pltpu
from jax.experimental.pallas import tpu_sc as plsc

N_LEADS = 12
D_MODEL = 256
BATCH = 16384
BLOCK_B = 8192


def _sc_gather(encoding_weight, pos_i32):
    mesh = plsc.VectorSubcoreMesh(core_axis_name="c", subcore_axis_name="s")

    @functools.partial(
        pl.kernel,
        mesh=mesh,
        out_type=jax.ShapeDtypeStruct((16, D_MODEL), jnp.float32),
        scratch_types=[
            pltpu.VMEM((16,), jnp.int32),
            pltpu.VMEM((16, D_MODEL), jnp.float32),
            pltpu.SemaphoreType.DMA,
        ],
    )
    def _k(w_hbm, pos_hbm, out_hbm, idx_v, rows_v, gsem):
        wid = lax.axis_index("s") * 2 + lax.axis_index("c")

        @pl.when(wid == 0)
        def _():
            pltpu.sync_copy(pos_hbm, idx_v)
            pltpu.async_copy(w_hbm.at[idx_v], rows_v, gsem).wait()
            pltpu.sync_copy(rows_v, out_hbm)

    return _k(encoding_weight, pos_i32)


def _add_body(enc_ref, x_ref, o_ref):
    lead = pl.program_id(0)
    enc = enc_ref[lead, :]
    o_ref[...] = x_ref[...] + enc[None, None, :]


def kernel(x, encoding_weight, positions):
    pos16 = jnp.zeros((16,), jnp.int32).at[:N_LEADS].set(
        positions.astype(jnp.int32))
    pos_enc = _sc_gather(encoding_weight, pos16)[:N_LEADS]
    x_t = jnp.transpose(x, (1, 0, 2))
    out_t = pl.pallas_call(
        _add_body,
        grid=(N_LEADS, BATCH // BLOCK_B),
        in_specs=[
            pl.BlockSpec((N_LEADS, D_MODEL), lambda l, i: (0, 0)),
            pl.BlockSpec((1, BLOCK_B, D_MODEL), lambda l, i: (l, i, 0)),
        ],
        out_specs=pl.BlockSpec((1, BLOCK_B, D_MODEL), lambda l, i: (l, i, 0)),
        out_shape=jax.ShapeDtypeStruct((N_LEADS, BATCH, D_MODEL), jnp.float32),
    )(pos_enc, x_t)
    return jnp.transpose(out_t, (1, 0, 2))
